# baseline (speedup 1.0000x reference)
import jax
import jax.numpy as jnp
from jax.experimental import pallas as pl
from jax.experimental.pallas import tpu as pltpu

N = 325
NU = 64
IN_DIM = 2
IS = IN_DIM + NU
ALPHA = 0.05
NUM_MAT = 5


def _prep_w(W, out_dim):
    a = ALPHA
    coef = jnp.array([1.0, 1.0, (1 - a) * a, (1 - a) ** 2 * a, (1 - a) ** 3],
                     dtype=jnp.float32)
    Wp = W.reshape(IS, NUM_MAT, out_dim).transpose(1, 0, 2) * coef[:, None, None]
    return Wp.reshape(NUM_MAT * IS, out_dim)


def _body(x1_ref, adp_ref, sup_ref, wru_ref, wc_ref, out_ref):
    adp = adp_ref[0]
    sup = sup_ref[...]
    x1 = x1_ref[0]
    hx = x1[:, IN_DIM:]

    def hops(x):
        s = jnp.dot(sup, x, preferred_element_type=jnp.float32)
        y1 = jnp.dot(adp, x, preferred_element_type=jnp.float32)
        y2 = jnp.dot(adp, y1, preferred_element_type=jnp.float32)
        y3 = jnp.dot(adp, y2, preferred_element_type=jnp.float32)
        return jnp.concatenate([x, s, y1, y2, y3], axis=1)

    ru = jax.nn.sigmoid(
        jnp.dot(hops(x1), wru_ref[...], preferred_element_type=jnp.float32))
    r = ru[:, :NU]
    u = ru[:, NU:]
    x2 = jnp.concatenate([x1[:, :IN_DIM], r * hx], axis=1)
    c = jnp.tanh(
        jnp.dot(hops(x2), wc_ref[...], preferred_element_type=jnp.float32))
    out_ref[0] = u * hx + (1.0 - u) * c


def kernel(inputs, hx, time_axis, adp, support, W_ru, W_c):
    B = inputs.shape[0]
    x1 = jnp.concatenate(
        [inputs.reshape(B, N, IN_DIM), hx.reshape(B, N, NU)], axis=2)
    wru = _prep_w(W_ru, 2 * NU)
    wc = _prep_w(W_c, NU)
    out = pl.pallas_call(
        _body,
        grid=(B,),
        in_specs=[
            pl.BlockSpec((1, N, IS), lambda b: (b, 0, 0)),
            pl.BlockSpec((1, N, N), lambda b: (b, 0, 0)),
            pl.BlockSpec((N, N), lambda b: (0, 0)),
            pl.BlockSpec((NUM_MAT * IS, 2 * NU), lambda b: (0, 0)),
            pl.BlockSpec((NUM_MAT * IS, NU), lambda b: (0, 0)),
        ],
        out_specs=pl.BlockSpec((1, N, NU), lambda b: (b, 0, 0)),
        out_shape=jax.ShapeDtypeStruct((B, N, NU), jnp.float32),
        compiler_params=pltpu.CompilerParams(
            dimension_semantics=("parallel",)),
    )(x1, adp, support, wru, wc)
    return out.reshape(B, N * NU)


# speedup vs baseline: 1.1904x; 1.1904x over previous
import jax
import jax.numpy as jnp
from jax import lax
from jax.experimental import pallas as pl
from jax.experimental.pallas import tpu as pltpu

N = 325
NU = 64
IN_DIM = 2
IS = IN_DIM + NU
ISP = 72
ALPHA = 0.05
NUM_MAT = 5


def _prep_w(W, out_dim):
    a = ALPHA
    coef = jnp.array([1.0, 1.0, (1 - a) * a, (1 - a) ** 2 * a, (1 - a) ** 3],
                     dtype=jnp.float32)
    Wp = W.reshape(IS, NUM_MAT, out_dim) * coef[None, :, None]
    Wp = jnp.concatenate(
        [Wp[IN_DIM:], Wp[:IN_DIM],
         jnp.zeros((ISP - IS, NUM_MAT, out_dim), jnp.float32)], axis=0)
    return Wp.transpose(1, 0, 2).reshape(NUM_MAT * ISP, out_dim).T


def _mmt(a, b):
    return lax.dot_general(a, b, (((1,), (1,)), ((), ())),
                           preferred_element_type=jnp.float32)


def _body(xt_ref, adp_ref, supt_ref, wrut_ref, wct_ref, out_ref):
    adp = adp_ref[0]
    supt = supt_ref[...]
    xt = xt_ref[0]
    hxt = xt[:NU]

    def hops(x):
        s = jnp.dot(x, supt, preferred_element_type=jnp.float32)
        y1 = _mmt(x, adp)
        y2 = _mmt(y1, adp)
        y3 = _mmt(y2, adp)
        return jnp.concatenate([x, s, y1, y2, y3], axis=0)

    ru = jax.nn.sigmoid(
        jnp.dot(wrut_ref[...], hops(xt), preferred_element_type=jnp.float32))
    r = ru[:NU]
    u = ru[NU:]
    x2 = jnp.concatenate([r * hxt, xt[NU:ISP]], axis=0)
    c = jnp.tanh(
        jnp.dot(wct_ref[...], hops(x2), preferred_element_type=jnp.float32))
    out_ref[0] = u * hxt + (1.0 - u) * c


def kernel(inputs, hx, time_axis, adp, support, W_ru, W_c):
    B = inputs.shape[0]
    xt = jnp.concatenate(
        [hx.reshape(B, N, NU), inputs.reshape(B, N, IN_DIM),
         jnp.zeros((B, N, ISP - IS), jnp.float32)], axis=2)
    xt = xt.transpose(0, 2, 1)
    wrut = _prep_w(W_ru, 2 * NU)
    wct = _prep_w(W_c, NU)
    out = pl.pallas_call(
        _body,
        grid=(B,),
        in_specs=[
            pl.BlockSpec((1, ISP, N), lambda b: (b, 0, 0)),
            pl.BlockSpec((1, N, N), lambda b: (b, 0, 0)),
            pl.BlockSpec((N, N), lambda b: (0, 0)),
            pl.BlockSpec((2 * NU, NUM_MAT * ISP), lambda b: (0, 0)),
            pl.BlockSpec((NU, NUM_MAT * ISP), lambda b: (0, 0)),
        ],
        out_specs=pl.BlockSpec((1, NU, N), lambda b: (b, 0, 0)),
        out_shape=jax.ShapeDtypeStruct((B, NU, N), jnp.float32),
        compiler_params=pltpu.CompilerParams(
            dimension_semantics=("parallel",)),
    )(xt, adp, support.T, wrut, wct)
    return out.transpose(0, 2, 1).reshape(B, N * NU)


# speedup vs baseline: 1.2274x; 1.0311x over previous
import jax
import jax.numpy as jnp
from jax import lax
from jax.experimental import pallas as pl
from jax.experimental.pallas import tpu as pltpu

N = 325
NU = 64
IN_DIM = 2
IS = IN_DIM + NU
ISP = 72
ALPHA = 0.05
NUM_MAT = 5


def _prep_w(W, out_dim):
    a = ALPHA
    coef = jnp.array([1.0, 1.0, (1 - a) * a, (1 - a) ** 2 * a, (1 - a) ** 3],
                     dtype=jnp.float32)
    Wp = W.reshape(IS, NUM_MAT, out_dim) * coef[None, :, None]
    Wp = jnp.concatenate(
        [Wp[IN_DIM:], Wp[:IN_DIM],
         jnp.zeros((ISP - IS, NUM_MAT, out_dim), jnp.float32)], axis=0)
    return Wp.transpose(1, 0, 2).reshape(NUM_MAT * ISP, out_dim).T


def _mmt(a, b):
    return lax.dot_general(a, b, (((1,), (1,)), ((), ())),
                           preferred_element_type=jnp.float32)


BB = 2


def _body(xt_ref, adp_ref, supt_ref, wrut_ref, wct_ref, out_ref):
    supt = supt_ref[...]
    for j in range(BB):
        adp = adp_ref[j]
        xt = xt_ref[j]
        hxt = xt[:NU]

        def hops(x):
            s = jnp.dot(x, supt, preferred_element_type=jnp.float32)
            y1 = _mmt(x, adp)
            y2 = _mmt(y1, adp)
            y3 = _mmt(y2, adp)
            return jnp.concatenate([x, s, y1, y2, y3], axis=0)

        ru = jax.nn.sigmoid(
            jnp.dot(wrut_ref[...], hops(xt),
                    preferred_element_type=jnp.float32))
        r = ru[:NU]
        u = ru[NU:]
        x2 = jnp.concatenate([r * hxt, xt[NU:ISP]], axis=0)
        c = jnp.tanh(
            jnp.dot(wct_ref[...], hops(x2),
                    preferred_element_type=jnp.float32))
        out_ref[j] = u * hxt + (1.0 - u) * c


def kernel(inputs, hx, time_axis, adp, support, W_ru, W_c):
    B = inputs.shape[0]
    xt = jnp.concatenate(
        [hx.reshape(B, N, NU), inputs.reshape(B, N, IN_DIM),
         jnp.zeros((B, N, ISP - IS), jnp.float32)], axis=2)
    xt = xt.transpose(0, 2, 1)
    wrut = _prep_w(W_ru, 2 * NU)
    wct = _prep_w(W_c, NU)
    out = pl.pallas_call(
        _body,
        grid=(B // BB,),
        in_specs=[
            pl.BlockSpec((BB, ISP, N), lambda b: (b, 0, 0)),
            pl.BlockSpec((BB, N, N), lambda b: (b, 0, 0)),
            pl.BlockSpec((N, N), lambda b: (0, 0)),
            pl.BlockSpec((2 * NU, NUM_MAT * ISP), lambda b: (0, 0)),
            pl.BlockSpec((NU, NUM_MAT * ISP), lambda b: (0, 0)),
        ],
        out_specs=pl.BlockSpec((BB, NU, N), lambda b: (b, 0, 0)),
        out_shape=jax.ShapeDtypeStruct((B, NU, N), jnp.float32),
        compiler_params=pltpu.CompilerParams(
            dimension_semantics=("parallel",)),
    )(xt, adp, support.T, wrut, wct)
    return out.transpose(0, 2, 1).reshape(B, N * NU)


# speedup vs baseline: 1.3747x; 1.1200x over previous
import jax
import jax.numpy as jnp
from jax import lax
from jax.experimental import pallas as pl
from jax.experimental.pallas import tpu as pltpu

N = 325
NU = 64
IN_DIM = 2
IS = IN_DIM + NU
ISP = 72
ALPHA = 0.05
NUM_MAT = 5
BB = 2


def _prep_w(W, out_dim):
    a = ALPHA
    coef = jnp.array([1.0, 1.0, (1 - a) * a, (1 - a) ** 2 * a, (1 - a) ** 3],
                     dtype=jnp.float32)
    Wp = W.reshape(IS, NUM_MAT, out_dim) * coef[None, :, None]
    Wp = jnp.concatenate(
        [Wp[IN_DIM:], Wp[:IN_DIM],
         jnp.zeros((ISP - IS, NUM_MAT, out_dim), jnp.float32)], axis=0)
    return Wp.transpose(1, 0, 2).reshape(NUM_MAT * ISP, out_dim).T


def _mmt(a, b):
    return lax.dot_general(a, b, (((1,), (1,)), ((), ())),
                           preferred_element_type=jnp.float32)


def _body(inp_ref, hx_ref, adp_ref, supt_ref, wrut_ref, wct_ref, out_ref):
    supt = supt_ref[...]
    for j in range(BB):
        adp = adp_ref[j]
        hxt = hx_ref[j].T
        inpt = jnp.pad(inp_ref[j].T, ((0, ISP - IS), (0, 0)))
        xt = jnp.concatenate([hxt, inpt], axis=0)

        def hops(x):
            s = jnp.dot(x, supt, preferred_element_type=jnp.float32)
            y1 = _mmt(x, adp)
            y2 = _mmt(y1, adp)
            y3 = _mmt(y2, adp)
            return jnp.concatenate([x, s, y1, y2, y3], axis=0)

        ru = jax.nn.sigmoid(
            jnp.dot(wrut_ref[...], hops(xt),
                    preferred_element_type=jnp.float32))
        r = ru[:NU]
        u = ru[NU:]
        x2 = jnp.concatenate([r * hxt, xt[NU:ISP]], axis=0)
        c = jnp.tanh(
            jnp.dot(wct_ref[...], hops(x2),
                    preferred_element_type=jnp.float32))
        out_ref[j] = (u * hxt + (1.0 - u) * c).T


def kernel(inputs, hx, time_axis, adp, support, W_ru, W_c):
    B = inputs.shape[0]
    wrut = _prep_w(W_ru, 2 * NU)
    wct = _prep_w(W_c, NU)
    out = pl.pallas_call(
        _body,
        grid=(B // BB,),
        in_specs=[
            pl.BlockSpec((BB, N, IN_DIM), lambda b: (b, 0, 0)),
            pl.BlockSpec((BB, N, NU), lambda b: (b, 0, 0)),
            pl.BlockSpec((BB, N, N), lambda b: (b, 0, 0)),
            pl.BlockSpec((N, N), lambda b: (0, 0)),
            pl.BlockSpec((2 * NU, NUM_MAT * ISP), lambda b: (0, 0)),
            pl.BlockSpec((NU, NUM_MAT * ISP), lambda b: (0, 0)),
        ],
        out_specs=pl.BlockSpec((BB, N, NU), lambda b: (b, 0, 0)),
        out_shape=jax.ShapeDtypeStruct((B, N, NU), jnp.float32),
        compiler_params=pltpu.CompilerParams(
            dimension_semantics=("parallel",)),
    )(inputs.reshape(B, N, IN_DIM), hx.reshape(B, N, NU), adp,
      support.T, wrut, wct)
    return out.reshape(B, N * NU)


# speedup vs baseline: 1.4165x; 1.0304x over previous
import jax
import jax.numpy as jnp
from jax import lax
from jax.experimental import pallas as pl
from jax.experimental.pallas import tpu as pltpu

N = 325
NU = 64
IN_DIM = 2
IS = IN_DIM + NU
ISP = 72
ALPHA = 0.05
NUM_MAT = 5
BB = 4


def _prep_w(W, out_dim):
    a = ALPHA
    coef = jnp.array([1.0, 1.0, (1 - a) * a, (1 - a) ** 2 * a, (1 - a) ** 3],
                     dtype=jnp.float32)
    Wp = W.reshape(IS, NUM_MAT, out_dim) * coef[None, :, None]
    Wp = jnp.concatenate(
        [Wp[IN_DIM:], Wp[:IN_DIM],
         jnp.zeros((ISP - IS, NUM_MAT, out_dim), jnp.float32)], axis=0)
    return Wp.transpose(1, 0, 2).reshape(NUM_MAT * ISP, out_dim).T


def _mmt(a, b):
    return lax.dot_general(a, b, (((1,), (1,)), ((), ())),
                           preferred_element_type=jnp.float32)


def _body(inp_ref, hx_ref, adp_ref, supt_ref, wrut_ref, wct_ref, out_ref):
    supt = supt_ref[...]
    for j in range(BB):
        adp = adp_ref[j]
        hxt = hx_ref[j].T
        inpt = jnp.pad(inp_ref[j].T, ((0, ISP - IS), (0, 0)))
        xt = jnp.concatenate([hxt, inpt], axis=0)

        def hops(x):
            s = jnp.dot(x, supt, preferred_element_type=jnp.float32)
            y1 = _mmt(x, adp)
            y2 = _mmt(y1, adp)
            y3 = _mmt(y2, adp)
            return jnp.concatenate([x, s, y1, y2, y3], axis=0)

        ru = jax.nn.sigmoid(
            jnp.dot(wrut_ref[...], hops(xt),
                    preferred_element_type=jnp.float32))
        r = ru[:NU]
        u = ru[NU:]
        x2 = jnp.concatenate([r * hxt, xt[NU:ISP]], axis=0)
        c = jnp.tanh(
            jnp.dot(wct_ref[...], hops(x2),
                    preferred_element_type=jnp.float32))
        out_ref[j] = (u * hxt + (1.0 - u) * c).T


def kernel(inputs, hx, time_axis, adp, support, W_ru, W_c):
    B = inputs.shape[0]
    wrut = _prep_w(W_ru, 2 * NU)
    wct = _prep_w(W_c, NU)
    out = pl.pallas_call(
        _body,
        grid=(B // BB,),
        in_specs=[
            pl.BlockSpec((BB, N, IN_DIM), lambda b: (b, 0, 0)),
            pl.BlockSpec((BB, N, NU), lambda b: (b, 0, 0)),
            pl.BlockSpec((BB, N, N), lambda b: (b, 0, 0)),
            pl.BlockSpec((N, N), lambda b: (0, 0)),
            pl.BlockSpec((2 * NU, NUM_MAT * ISP), lambda b: (0, 0)),
            pl.BlockSpec((NU, NUM_MAT * ISP), lambda b: (0, 0)),
        ],
        out_specs=pl.BlockSpec((BB, N, NU), lambda b: (b, 0, 0)),
        out_shape=jax.ShapeDtypeStruct((B, N, NU), jnp.float32),
        compiler_params=pltpu.CompilerParams(
            dimension_semantics=("parallel",)),
    )(inputs.reshape(B, N, IN_DIM), hx.reshape(B, N, NU), adp,
      support.T, wrut, wct)
    return out.reshape(B, N * NU)
